# Initial kernel scaffold; baseline (speedup 1.0000x reference)
import jax
import jax.numpy as jnp
from jax.experimental import pallas as pl


def kernel(idx, root_bond_pos, embedding, bond_pos_tensors, one_hot_pos, W_root, b_root):
    raise NotImplementedError("write your pallas kernel here")


# speedup vs baseline: 2.8583x; 2.8583x over previous
import functools

import jax
import jax.numpy as jnp
from jax import lax
from jax.experimental import pallas as pl
from jax.experimental.pallas import tpu as pltpu
from jax.experimental.pallas import tpu_sc as plsc

VOCAB = 100000
NODE_DIM = 128
MAX_BOND = 16

_ROWS_PER_BLOCK = 2048
_NC = 2
_NS = 16
_NW = _NC * _NS
_CHUNK = 128


def _proj_body(m_ref, w_ref, b_ref, o_ref):
    i = pl.program_id(0)
    rows = lax.broadcasted_iota(jnp.int32, (_ROWS_PER_BLOCK, 1), 0) + i * _ROWS_PER_BLOCK
    bias = jnp.where(rows < VOCAB, b_ref[...], 0.0)
    o_ref[...] = jnp.dot(m_ref[...], w_ref[...],
                         preferred_element_type=jnp.float32) + bias


def _project_table(m, w, b2):
    mpad = m.shape[0]
    grid = mpad // _ROWS_PER_BLOCK
    return pl.pallas_call(
        _proj_body,
        grid=(grid,),
        in_specs=[
            pl.BlockSpec((_ROWS_PER_BLOCK, MAX_BOND), lambda i: (i, 0)),
            pl.BlockSpec((MAX_BOND, NODE_DIM), lambda i: (0, 0)),
            pl.BlockSpec((1, NODE_DIM), lambda i: (0, 0)),
        ],
        out_specs=pl.BlockSpec((_ROWS_PER_BLOCK, NODE_DIM), lambda i: (i, 0)),
        out_shape=jax.ShapeDtypeStruct((mpad, NODE_DIM), jnp.float32),
    )(m, w, b2)


def _make_sc_kernel(n):
    per_w = n // _NW
    nchunks = per_w // _CHUNK
    mesh = plsc.VectorSubcoreMesh(core_axis_name="c", subcore_axis_name="s",
                                  num_cores=_NC, num_subcores=_NS)

    @functools.partial(
        pl.kernel,
        out_type=jax.ShapeDtypeStruct((n, NODE_DIM), jnp.float32),
        mesh=mesh,
        compiler_params=pltpu.CompilerParams(needs_layout_passes=False),
        scratch_types=[
            pltpu.VMEM((_CHUNK,), jnp.int32),
            pltpu.VMEM((_CHUNK,), jnp.int32),
            pltpu.VMEM((_CHUNK, NODE_DIM), jnp.float32),
            pltpu.VMEM((_CHUNK, NODE_DIM), jnp.float32),
            pltpu.VMEM((24, NODE_DIM), jnp.float32),
            pltpu.SemaphoreType.DMA,
            pltpu.SemaphoreType.DMA,
        ],
    )
    def sc_kernel(idx_hbm, rbp_hbm, emb_hbm, c_hbm, out_hbm,
                  idx_v, rbp_v, ebuf, wbuf, pbuf, sem_e, sem_w):
        wid = lax.axis_index("s") * _NC + lax.axis_index("c")
        pltpu.sync_copy(c_hbm.at[pl.ds(VOCAB, 24)], pbuf)

        def chunk_body(k, carry):
            base = wid * per_w + k * _CHUNK
            pltpu.sync_copy(idx_hbm.at[pl.ds(base, _CHUNK)], idx_v)
            pltpu.sync_copy(rbp_hbm.at[pl.ds(base, _CHUNK)], rbp_v)
            ce = pltpu.async_copy(emb_hbm.at[idx_v], ebuf, sem_e)
            cw = pltpu.async_copy(c_hbm.at[idx_v], wbuf, sem_w)
            ce.wait()
            cw.wait()

            def tok_body(t, tc):
                rsp = plsc.load_gather(rbp_v, [jnp.full((16,), t, jnp.int32)])
                for c in range(NODE_DIM // 16):
                    col = lax.iota(jnp.int32, 16) + c * 16
                    pv = plsc.load_gather(pbuf, [rsp, col])
                    ev = ebuf[t, pl.ds(c * 16, 16)]
                    wv = wbuf[t, pl.ds(c * 16, 16)]
                    wbuf[t, pl.ds(c * 16, 16)] = ev * (wv + pv)
                return tc

            lax.fori_loop(0, _CHUNK, tok_body, 0)
            pltpu.sync_copy(wbuf, out_hbm.at[pl.ds(base, _CHUNK)])
            return carry

        lax.fori_loop(0, nchunks, chunk_body, 0)

    return sc_kernel


def kernel(idx, root_bond_pos, embedding, bond_pos_tensors, one_hot_pos,
           W_root, b_root):
    b, l = idx.shape
    n = b * l
    idx_f = idx.reshape(n).astype(jnp.int32)
    rbp_f = root_bond_pos.reshape(n).astype(jnp.int32)

    m = jnp.concatenate([bond_pos_tensors[:VOCAB], one_hot_pos], axis=0)
    mpad = ((m.shape[0] + _ROWS_PER_BLOCK - 1) // _ROWS_PER_BLOCK) * _ROWS_PER_BLOCK
    m = jnp.pad(m, ((0, mpad - m.shape[0]), (0, 0)))
    c = _project_table(m, W_root.astype(jnp.float32),
                       b_root.reshape(1, NODE_DIM).astype(jnp.float32))

    out = _make_sc_kernel(n)(idx_f, rbp_f, embedding, c)
    return out.reshape(b, l, NODE_DIM)


# speedup vs baseline: 3.2262x; 1.1287x over previous
import functools

import jax
import jax.numpy as jnp
from jax import lax
from jax.experimental import pallas as pl
from jax.experimental.pallas import tpu as pltpu
from jax.experimental.pallas import tpu_sc as plsc

VOCAB = 100000
NODE_DIM = 128
MAX_BOND = 16

_ROWS_PER_BLOCK = 2048
_NC = 2
_NS = 16
_NW = _NC * _NS
_CHUNK = 128


def _proj_body(m_ref, w_ref, b_ref, o_ref):
    i = pl.program_id(0)
    rows = lax.broadcasted_iota(jnp.int32, (_ROWS_PER_BLOCK, 1), 0) + i * _ROWS_PER_BLOCK
    bias = jnp.where(rows < VOCAB, b_ref[...], 0.0)
    o_ref[...] = jnp.dot(m_ref[...], w_ref[...],
                         preferred_element_type=jnp.float32) + bias


def _project_table(m, w, b2):
    mpad = m.shape[0]
    grid = mpad // _ROWS_PER_BLOCK
    return pl.pallas_call(
        _proj_body,
        grid=(grid,),
        in_specs=[
            pl.BlockSpec((_ROWS_PER_BLOCK, MAX_BOND), lambda i: (i, 0)),
            pl.BlockSpec((MAX_BOND, NODE_DIM), lambda i: (0, 0)),
            pl.BlockSpec((1, NODE_DIM), lambda i: (0, 0)),
        ],
        out_specs=pl.BlockSpec((_ROWS_PER_BLOCK, NODE_DIM), lambda i: (i, 0)),
        out_shape=jax.ShapeDtypeStruct((mpad, NODE_DIM), jnp.float32),
    )(m, w, b2)


def _make_sc_kernel(n):
    per_w = n // _NW
    nchunks = per_w // _CHUNK
    mesh = plsc.VectorSubcoreMesh(core_axis_name="c", subcore_axis_name="s",
                                  num_cores=_NC, num_subcores=_NS)

    @functools.partial(
        pl.kernel,
        out_type=jax.ShapeDtypeStruct((n, NODE_DIM), jnp.float32),
        mesh=mesh,
        compiler_params=pltpu.CompilerParams(needs_layout_passes=False),
        scratch_types=[
            pltpu.VMEM((2, _CHUNK), jnp.int32),
            pltpu.VMEM((2, _CHUNK), jnp.int32),
            pltpu.VMEM((2, _CHUNK, NODE_DIM), jnp.float32),
            pltpu.VMEM((2, _CHUNK, NODE_DIM), jnp.float32),
            pltpu.VMEM((24, NODE_DIM), jnp.float32),
            pltpu.SemaphoreType.DMA,
            pltpu.SemaphoreType.DMA,
            pltpu.SemaphoreType.DMA,
            pltpu.SemaphoreType.DMA,
        ],
    )
    def sc_kernel(idx_hbm, rbp_hbm, emb_hbm, c_hbm, out_hbm,
                  idx_v, rbp_v, ebuf, wbuf, pbuf, sem_e, sem_w, sem_o0, sem_o1):
        wid = lax.axis_index("s") * _NC + lax.axis_index("c")
        w0 = wid * per_w
        sem_o = [sem_o0, sem_o1]
        pltpu.sync_copy(c_hbm.at[pl.ds(VOCAB, 24)], pbuf)
        cols = [lax.iota(jnp.int32, 16) + c * 16 for c in range(NODE_DIM // 16)]

        def stage(k, s):
            base = w0 + k * _CHUNK
            pltpu.sync_copy(idx_hbm.at[pl.ds(base, _CHUNK)], idx_v.at[s])
            pltpu.sync_copy(rbp_hbm.at[pl.ds(base, _CHUNK)], rbp_v.at[s])
            pltpu.async_copy(emb_hbm.at[idx_v.at[s]], ebuf.at[s], sem_e)
            pltpu.async_copy(c_hbm.at[idx_v.at[s]], wbuf.at[s], sem_w)

        def wait_gathers(s):
            pltpu.make_async_copy(emb_hbm.at[idx_v.at[s]], ebuf.at[s], sem_e).wait()
            pltpu.make_async_copy(c_hbm.at[idx_v.at[s]], wbuf.at[s], sem_w).wait()

        def wait_writeback(s):
            pltpu.make_async_copy(
                wbuf.at[s], out_hbm.at[pl.ds(w0, _CHUNK)], sem_o[s]).wait()

        def compute(s):
            def tok_body(t, carry):
                rsp = plsc.load_gather(rbp_v.at[s], [jnp.full((16,), t, jnp.int32)])
                for c in range(NODE_DIM // 16):
                    pv = plsc.load_gather(pbuf, [rsp, cols[c]])
                    ev = ebuf[s, t, pl.ds(c * 16, 16)]
                    wv = wbuf[s, t, pl.ds(c * 16, 16)]
                    wbuf[s, t, pl.ds(c * 16, 16)] = ev * (wv + pv)
                return carry

            lax.fori_loop(0, _CHUNK, tok_body, 0)

        stage(0, 0)

        @pl.loop(0, nchunks, step=2)
        def _(g):
            for b in range(2):
                k = g + b
                s = b
                o = 1 - b

                wait_gathers(s)

                @pl.when(k + 1 < nchunks)
                def _():
                    @pl.when(k >= 1)
                    def _():
                        wait_writeback(o)

                    stage(k + 1, o)

                compute(s)
                pltpu.async_copy(
                    wbuf.at[s], out_hbm.at[pl.ds(w0 + k * _CHUNK, _CHUNK)],
                    sem_o[s])

        wait_writeback((nchunks - 2) % 2)
        wait_writeback((nchunks - 1) % 2)

    return sc_kernel


def kernel(idx, root_bond_pos, embedding, bond_pos_tensors, one_hot_pos,
           W_root, b_root):
    b, l = idx.shape
    n = b * l
    idx_f = idx.reshape(n).astype(jnp.int32)
    rbp_f = root_bond_pos.reshape(n).astype(jnp.int32)

    m = jnp.concatenate([bond_pos_tensors[:VOCAB], one_hot_pos], axis=0)
    mpad = ((m.shape[0] + _ROWS_PER_BLOCK - 1) // _ROWS_PER_BLOCK) * _ROWS_PER_BLOCK
    m = jnp.pad(m, ((0, mpad - m.shape[0]), (0, 0)))
    c = _project_table(m, W_root.astype(jnp.float32),
                       b_root.reshape(1, NODE_DIM).astype(jnp.float32))

    out = _make_sc_kernel(n)(idx_f, rbp_f, embedding, c)
    return out.reshape(b, l, NODE_DIM)


# speedup vs baseline: 3.4573x; 1.0716x over previous
import functools

import jax
import jax.numpy as jnp
from jax import lax
from jax.experimental import pallas as pl
from jax.experimental.pallas import tpu as pltpu
from jax.experimental.pallas import tpu_sc as plsc

VOCAB = 100000
NODE_DIM = 128
MAX_BOND = 16

_ROWS_PER_BLOCK = 2048
_NC = 2
_NS = 16
_NW = _NC * _NS
_CHUNK = 128


def _proj_body(m_ref, w_ref, b_ref, o_ref):
    i = pl.program_id(0)
    rows = lax.broadcasted_iota(jnp.int32, (_ROWS_PER_BLOCK, 1), 0) + i * _ROWS_PER_BLOCK
    bias = jnp.where(rows < VOCAB, b_ref[...], 0.0)
    o_ref[...] = jnp.dot(m_ref[...], w_ref[...],
                         preferred_element_type=jnp.float32) + bias


def _project_table(m, w, b2):
    mpad = m.shape[0]
    grid = mpad // _ROWS_PER_BLOCK
    return pl.pallas_call(
        _proj_body,
        grid=(grid,),
        in_specs=[
            pl.BlockSpec((_ROWS_PER_BLOCK, MAX_BOND), lambda i: (i, 0)),
            pl.BlockSpec((MAX_BOND, NODE_DIM), lambda i: (0, 0)),
            pl.BlockSpec((1, NODE_DIM), lambda i: (0, 0)),
        ],
        out_specs=pl.BlockSpec((_ROWS_PER_BLOCK, NODE_DIM), lambda i: (i, 0)),
        out_shape=jax.ShapeDtypeStruct((mpad, NODE_DIM), jnp.float32),
    )(m, w, b2)


def _make_sc_kernel(n):
    per_w = n // _NW
    nchunks = per_w // _CHUNK
    mesh = plsc.VectorSubcoreMesh(core_axis_name="c", subcore_axis_name="s",
                                  num_cores=_NC, num_subcores=_NS)

    @functools.partial(
        pl.kernel,
        out_type=jax.ShapeDtypeStruct((n, NODE_DIM), jnp.float32),
        mesh=mesh,
        compiler_params=pltpu.CompilerParams(needs_layout_passes=False,
                                             use_tc_tiling_on_sc=True),
        scratch_types=[
            pltpu.VMEM((per_w,), jnp.int32),
            pltpu.VMEM((per_w,), jnp.int32),
            pltpu.VMEM((2, _CHUNK, NODE_DIM), jnp.float32),
            pltpu.VMEM((2, _CHUNK, NODE_DIM), jnp.float32),
            pltpu.VMEM((24, NODE_DIM), jnp.float32),
            pltpu.SemaphoreType.DMA,
            pltpu.SemaphoreType.DMA,
            pltpu.SemaphoreType.DMA,
            pltpu.SemaphoreType.DMA,
        ],
    )
    def sc_kernel(idx_hbm, rbp_hbm, emb_hbm, c_hbm, out_hbm,
                  idx_v, rbp_v, ebuf, wbuf, pbuf, sem_e, sem_w, sem_o0, sem_o1):
        wid = lax.axis_index("s") * _NC + lax.axis_index("c")
        w0 = wid * per_w
        sem_o = [sem_o0, sem_o1]
        pltpu.sync_copy(idx_hbm.at[pl.ds(w0, per_w)], idx_v)
        pltpu.sync_copy(rbp_hbm.at[pl.ds(w0, per_w)], rbp_v)
        pltpu.sync_copy(c_hbm.at[pl.ds(VOCAB, 24)], pbuf)
        cols = [lax.iota(jnp.int32, 16) + c * 16 for c in range(NODE_DIM // 16)]

        def stage(k, s):
            ii = idx_v.at[pl.ds(k * _CHUNK, _CHUNK)]
            pltpu.async_copy(emb_hbm.at[ii], ebuf.at[s], sem_e)
            pltpu.async_copy(c_hbm.at[ii], wbuf.at[s], sem_w)

        def wait_gathers(s):
            ii = idx_v.at[pl.ds(0, _CHUNK)]
            pltpu.make_async_copy(emb_hbm.at[ii], ebuf.at[s], sem_e).wait()
            pltpu.make_async_copy(c_hbm.at[ii], wbuf.at[s], sem_w).wait()

        def wait_writeback(s):
            pltpu.make_async_copy(
                wbuf.at[s], out_hbm.at[pl.ds(w0, _CHUNK)], sem_o[s]).wait()

        def compute(k, s):
            kc = k * _CHUNK

            def tok_body(t, carry):
                rsp = plsc.load_gather(rbp_v, [jnp.full((16,), kc + t, jnp.int32)])
                for c in range(NODE_DIM // 16):
                    pv = plsc.load_gather(pbuf, [rsp, cols[c]])
                    ev = ebuf[s, t, pl.ds(c * 16, 16)]
                    wv = wbuf[s, t, pl.ds(c * 16, 16)]
                    wbuf[s, t, pl.ds(c * 16, 16)] = ev * (wv + pv)
                return carry

            lax.fori_loop(0, _CHUNK, tok_body, 0)

        stage(0, 0)

        @pl.loop(0, nchunks, step=2)
        def _(g):
            for b in range(2):
                k = g + b
                s = b
                o = 1 - b

                wait_gathers(s)

                @pl.when(k + 1 < nchunks)
                def _():
                    @pl.when(k >= 1)
                    def _():
                        wait_writeback(o)

                    stage(k + 1, o)

                compute(k, s)
                pltpu.async_copy(
                    wbuf.at[s], out_hbm.at[pl.ds(w0 + k * _CHUNK, _CHUNK)],
                    sem_o[s])

        wait_writeback((nchunks - 2) % 2)
        wait_writeback((nchunks - 1) % 2)

    return sc_kernel


def kernel(idx, root_bond_pos, embedding, bond_pos_tensors, one_hot_pos,
           W_root, b_root):
    b, l = idx.shape
    n = b * l
    idx_f = idx.reshape(n).astype(jnp.int32)
    rbp_f = root_bond_pos.reshape(n).astype(jnp.int32)

    m = jnp.concatenate([bond_pos_tensors[:VOCAB], one_hot_pos], axis=0)
    mpad = ((m.shape[0] + _ROWS_PER_BLOCK - 1) // _ROWS_PER_BLOCK) * _ROWS_PER_BLOCK
    m = jnp.pad(m, ((0, mpad - m.shape[0]), (0, 0)))
    c = _project_table(m, W_root.astype(jnp.float32),
                       b_root.reshape(1, NODE_DIM).astype(jnp.float32))

    out = _make_sc_kernel(n)(idx_f, rbp_f, embedding, c)
    return out.reshape(b, l, NODE_DIM)


# speedup vs baseline: 5.2678x; 1.5237x over previous
import functools

import jax
import jax.numpy as jnp
from jax import lax
from jax.experimental import pallas as pl
from jax.experimental.pallas import tpu as pltpu
from jax.experimental.pallas import tpu_sc as plsc

VOCAB = 100000
NODE_DIM = 128
MAX_BOND = 16

_ROWS_PER_BLOCK = 2048
_NC = 2
_NS = 16
_NW = _NC * _NS
_CHUNK = 128


def _proj_body(mt_ref, oht_ref, w_ref, b_ref, c_ref, p_ref):
    i = pl.program_id(0)
    rows = lax.broadcasted_iota(jnp.int32, (_ROWS_PER_BLOCK, 1), 0) + i * _ROWS_PER_BLOCK
    bias = jnp.where(rows < VOCAB, b_ref[...], 0.0)
    dn = (((0,), (0,)), ((), ()))
    c_ref[...] = lax.dot_general(mt_ref[...], w_ref[...], dn,
                                 preferred_element_type=jnp.float32) + bias

    @pl.when(i == 0)
    def _():
        p_ref[...] = lax.dot_general(oht_ref[...], w_ref[...], dn,
                                     preferred_element_type=jnp.float32)


def _project_tables(mt, oht, w, b2, cpad):
    grid = cpad // _ROWS_PER_BLOCK
    return pl.pallas_call(
        _proj_body,
        grid=(grid,),
        in_specs=[
            pl.BlockSpec((MAX_BOND, _ROWS_PER_BLOCK), lambda i: (0, i)),
            pl.BlockSpec((MAX_BOND, 24), lambda i: (0, 0)),
            pl.BlockSpec((MAX_BOND, NODE_DIM), lambda i: (0, 0)),
            pl.BlockSpec((1, NODE_DIM), lambda i: (0, 0)),
        ],
        out_specs=[
            pl.BlockSpec((_ROWS_PER_BLOCK, NODE_DIM), lambda i: (i, 0)),
            pl.BlockSpec((24, NODE_DIM), lambda i: (0, 0)),
        ],
        out_shape=[
            jax.ShapeDtypeStruct((cpad, NODE_DIM), jnp.float32),
            jax.ShapeDtypeStruct((24, NODE_DIM), jnp.float32),
        ],
    )(mt, oht, w, b2)


def _make_sc_kernel(n):
    per_w = n // _NW
    nchunks = per_w // _CHUNK
    mesh = plsc.VectorSubcoreMesh(core_axis_name="c", subcore_axis_name="s",
                                  num_cores=_NC, num_subcores=_NS)

    @functools.partial(
        pl.kernel,
        out_type=jax.ShapeDtypeStruct((n, NODE_DIM), jnp.float32),
        mesh=mesh,
        compiler_params=pltpu.CompilerParams(needs_layout_passes=False,
                                             use_tc_tiling_on_sc=True),
        scratch_types=[
            pltpu.VMEM((per_w,), jnp.int32),
            pltpu.VMEM((per_w,), jnp.int32),
            pltpu.VMEM((2, _CHUNK, NODE_DIM), jnp.float32),
            pltpu.VMEM((2, _CHUNK, NODE_DIM), jnp.float32),
            pltpu.VMEM((24, NODE_DIM), jnp.float32),
            pltpu.SemaphoreType.DMA,
            pltpu.SemaphoreType.DMA,
            pltpu.SemaphoreType.DMA,
            pltpu.SemaphoreType.DMA,
        ],
    )
    def sc_kernel(idx_hbm, rbp_hbm, emb_hbm, c_hbm, p_hbm, out_hbm,
                  idx_v, rbp_v, ebuf, wbuf, pbuf, sem_e, sem_w, sem_o0, sem_o1):
        wid = lax.axis_index("s") * _NC + lax.axis_index("c")
        w0 = wid * per_w
        sem_o = [sem_o0, sem_o1]
        pltpu.sync_copy(idx_hbm.at[pl.ds(w0, per_w)], idx_v)
        pltpu.sync_copy(rbp_hbm.at[pl.ds(w0, per_w)], rbp_v)
        pltpu.sync_copy(p_hbm, pbuf)
        cols = [lax.iota(jnp.int32, 16) + c * 16 for c in range(NODE_DIM // 16)]

        def stage(k, s):
            ii = idx_v.at[pl.ds(k * _CHUNK, _CHUNK)]
            pltpu.async_copy(emb_hbm.at[ii], ebuf.at[s], sem_e)
            pltpu.async_copy(c_hbm.at[ii], wbuf.at[s], sem_w)

        def wait_gathers(s):
            ii = idx_v.at[pl.ds(0, _CHUNK)]
            pltpu.make_async_copy(emb_hbm.at[ii], ebuf.at[s], sem_e).wait()
            pltpu.make_async_copy(c_hbm.at[ii], wbuf.at[s], sem_w).wait()

        def wait_writeback(s):
            pltpu.make_async_copy(
                wbuf.at[s], out_hbm.at[pl.ds(w0, _CHUNK)], sem_o[s]).wait()

        def compute(k, s):
            kc = k * _CHUNK

            def tok_body(t, carry):
                rsp = plsc.load_gather(rbp_v, [jnp.full((16,), kc + t, jnp.int32)])
                for c in range(NODE_DIM // 16):
                    pv = plsc.load_gather(pbuf, [rsp, cols[c]])
                    ev = ebuf[s, t, pl.ds(c * 16, 16)]
                    wv = wbuf[s, t, pl.ds(c * 16, 16)]
                    wbuf[s, t, pl.ds(c * 16, 16)] = ev * (wv + pv)
                return carry

            lax.fori_loop(0, _CHUNK, tok_body, 0)

        stage(0, 0)

        @pl.loop(0, nchunks, step=2)
        def _(g):
            for b in range(2):
                k = g + b
                s = b
                o = 1 - b

                wait_gathers(s)

                @pl.when(k + 1 < nchunks)
                def _():
                    @pl.when(k >= 1)
                    def _():
                        wait_writeback(o)

                    stage(k + 1, o)

                compute(k, s)
                pltpu.async_copy(
                    wbuf.at[s], out_hbm.at[pl.ds(w0 + k * _CHUNK, _CHUNK)],
                    sem_o[s])

        wait_writeback((nchunks - 2) % 2)
        wait_writeback((nchunks - 1) % 2)

    return sc_kernel


def kernel(idx, root_bond_pos, embedding, bond_pos_tensors, one_hot_pos,
           W_root, b_root):
    b, l = idx.shape
    n = b * l
    idx_f = jnp.swapaxes(idx, 0, 1).reshape(n).astype(jnp.int32)
    rbp_f = jnp.swapaxes(root_bond_pos, 0, 1).reshape(n).astype(jnp.int32)

    cpad = ((bond_pos_tensors.shape[0] + _ROWS_PER_BLOCK - 1)
            // _ROWS_PER_BLOCK) * _ROWS_PER_BLOCK
    mt = bond_pos_tensors.T
    oht = jnp.pad(one_hot_pos.T, ((0, 0), (0, 24 - one_hot_pos.shape[0])))
    c, p = _project_tables(mt, oht, W_root.astype(jnp.float32),
                           b_root.reshape(1, NODE_DIM).astype(jnp.float32),
                           cpad)

    out = _make_sc_kernel(n)(idx_f, rbp_f, embedding, c, p)
    return jnp.swapaxes(out.reshape(l, b, NODE_DIM), 0, 1)


# speedup vs baseline: 11.8761x; 2.2545x over previous
import functools

import jax
import jax.numpy as jnp
from jax import lax
from jax.experimental import pallas as pl
from jax.experimental.pallas import tpu as pltpu
from jax.experimental.pallas import tpu_sc as plsc

VOCAB = 100000
NODE_DIM = 128
MAX_BOND = 16

_ROWS_PER_BLOCK = 2048
_NC = 2
_NS = 16
_NW = _NC * _NS
_CHUNK = 128


def _proj_body(mt_ref, oht_ref, w_ref, b_ref, c_ref, p_ref):
    i = pl.program_id(0)
    rows = lax.broadcasted_iota(jnp.int32, (_ROWS_PER_BLOCK, 1), 0) + i * _ROWS_PER_BLOCK
    bias = jnp.where(rows < VOCAB, b_ref[...], 0.0)
    dn = (((0,), (0,)), ((), ()))
    c_ref[...] = lax.dot_general(mt_ref[...], w_ref[...], dn,
                                 preferred_element_type=jnp.float32) + bias

    @pl.when(i == 0)
    def _():
        p_ref[...] = lax.dot_general(oht_ref[...], w_ref[...], dn,
                                     preferred_element_type=jnp.float32)


def _project_tables(mt, oht, w, b2, cpad):
    grid = cpad // _ROWS_PER_BLOCK
    return pl.pallas_call(
        _proj_body,
        grid=(grid,),
        in_specs=[
            pl.BlockSpec((MAX_BOND, _ROWS_PER_BLOCK), lambda i: (0, i)),
            pl.BlockSpec((MAX_BOND, 24), lambda i: (0, 0)),
            pl.BlockSpec((MAX_BOND, NODE_DIM), lambda i: (0, 0)),
            pl.BlockSpec((1, NODE_DIM), lambda i: (0, 0)),
        ],
        out_specs=[
            pl.BlockSpec((_ROWS_PER_BLOCK, NODE_DIM), lambda i: (i, 0)),
            pl.BlockSpec((24, NODE_DIM), lambda i: (0, 0)),
        ],
        out_shape=[
            jax.ShapeDtypeStruct((cpad, NODE_DIM), jnp.float32),
            jax.ShapeDtypeStruct((24, NODE_DIM), jnp.float32),
        ],
    )(mt, oht, w, b2)


def _make_sc_kernel(n):
    per_w = n // _NW
    nchunks = per_w // _CHUNK
    mesh = plsc.VectorSubcoreMesh(core_axis_name="c", subcore_axis_name="s",
                                  num_cores=_NC, num_subcores=_NS)

    @functools.partial(
        pl.kernel,
        out_type=jax.ShapeDtypeStruct((n, NODE_DIM), jnp.float32),
        mesh=mesh,
        compiler_params=pltpu.CompilerParams(needs_layout_passes=False,
                                             use_tc_tiling_on_sc=True),
        scratch_types=[
            pltpu.VMEM((per_w,), jnp.int32),
            pltpu.VMEM((per_w,), jnp.int32),
            pltpu.VMEM((2, _CHUNK, NODE_DIM), jnp.float32),
            pltpu.VMEM((2, _CHUNK, NODE_DIM), jnp.float32),
            pltpu.VMEM((24 * NODE_DIM,), jnp.float32),
            pltpu.SemaphoreType.DMA,
            pltpu.SemaphoreType.DMA,
            pltpu.SemaphoreType.DMA,
            pltpu.SemaphoreType.DMA,
        ],
    )
    def sc_kernel(idx_hbm, rbp_hbm, emb_hbm, c_hbm, p_hbm, out_hbm,
                  idx_v, rbp_v, ebuf, wbuf, pbuf, sem_e, sem_w, sem_o0, sem_o1):
        wid = lax.axis_index("s") * _NC + lax.axis_index("c")
        w0 = wid * per_w
        sem_o = [sem_o0, sem_o1]
        pltpu.sync_copy(idx_hbm.at[pl.ds(w0, per_w)], idx_v)
        pltpu.sync_copy(rbp_hbm.at[pl.ds(w0, per_w)], rbp_v)
        pltpu.sync_copy(p_hbm, pbuf)
        cols = [lax.iota(jnp.int32, 16) + c * 16 for c in range(NODE_DIM // 16)]

        def stage(k, s):
            ii = idx_v.at[pl.ds(k * _CHUNK, _CHUNK)]
            pltpu.async_copy(emb_hbm.at[ii], ebuf.at[s], sem_e)
            pltpu.async_copy(c_hbm.at[ii], wbuf.at[s], sem_w)

        def wait_gathers(s):
            ii = idx_v.at[pl.ds(0, _CHUNK)]
            pltpu.make_async_copy(emb_hbm.at[ii], ebuf.at[s], sem_e).wait()
            pltpu.make_async_copy(c_hbm.at[ii], wbuf.at[s], sem_w).wait()

        def wait_writeback(s):
            pltpu.make_async_copy(
                wbuf.at[s], out_hbm.at[pl.ds(w0, _CHUNK)], sem_o[s]).wait()

        def compute(k, s):
            kc = k * _CHUNK

            @plsc.parallel_loop(0, _CHUNK, unroll=8)
            def _(t):
                rsp = plsc.load_gather(rbp_v, [jnp.full((16,), kc + t, jnp.int32)])
                rb = rsp * NODE_DIM
                for c in range(NODE_DIM // 16):
                    pv = plsc.load_gather(pbuf, [rb + cols[c]])
                    ev = ebuf[s, t, pl.ds(c * 16, 16)]
                    wv = wbuf[s, t, pl.ds(c * 16, 16)]
                    wbuf[s, t, pl.ds(c * 16, 16)] = ev * (wv + pv)

        stage(0, 0)

        @pl.loop(0, nchunks, step=2)
        def _(g):
            for b in range(2):
                k = g + b
                s = b
                o = 1 - b

                wait_gathers(s)

                @pl.when(k + 1 < nchunks)
                def _():
                    @pl.when(k >= 1)
                    def _():
                        wait_writeback(o)

                    stage(k + 1, o)

                compute(k, s)
                pltpu.async_copy(
                    wbuf.at[s], out_hbm.at[pl.ds(w0 + k * _CHUNK, _CHUNK)],
                    sem_o[s])

        wait_writeback((nchunks - 2) % 2)
        wait_writeback((nchunks - 1) % 2)

    return sc_kernel


def kernel(idx, root_bond_pos, embedding, bond_pos_tensors, one_hot_pos,
           W_root, b_root):
    b, l = idx.shape
    n = b * l
    idx_f = jnp.swapaxes(idx, 0, 1).reshape(n).astype(jnp.int32)
    rbp_f = jnp.swapaxes(root_bond_pos, 0, 1).reshape(n).astype(jnp.int32)

    cpad = ((bond_pos_tensors.shape[0] + _ROWS_PER_BLOCK - 1)
            // _ROWS_PER_BLOCK) * _ROWS_PER_BLOCK
    mt = bond_pos_tensors.T
    oht = jnp.pad(one_hot_pos.T, ((0, 0), (0, 24 - one_hot_pos.shape[0])))
    c, p = _project_tables(mt, oht, W_root.astype(jnp.float32),
                           b_root.reshape(1, NODE_DIM).astype(jnp.float32),
                           cpad)

    out = _make_sc_kernel(n)(idx_f, rbp_f, embedding, c, p.reshape(-1))
    return jnp.swapaxes(out.reshape(l, b, NODE_DIM), 0, 1)
